# baseline (speedup 1.0000x reference)
import jax
import jax.numpy as jnp
from jax.experimental import pallas as pl
from jax.experimental.pallas import tpu as pltpu

E = 8
H = 1024
I_DIM = 512
IS_DIM = 512
T = 2048

def _silu_mul(g, u):
    return (g * jax.nn.sigmoid(g)) * u


def _moe_body(x_ref, gate_w_ref, wg_ref, wu_ref, wd_ref,
              swg_ref, swu_ref, swd_ref, out_ref, comb_ref, xb_ref):
    e = pl.program_id(0)

    @pl.when(e == 0)
    def _router():
        x32 = x_ref[...]
        logits = jax.lax.dot_general(
            x32, gate_w_ref[...], (((1,), (1,)), ((), ())),
            preferred_element_type=jnp.float32)
        m = jnp.max(logits, axis=-1, keepdims=True)
        p = jnp.exp(logits - m)
        p = p / jnp.sum(p, axis=-1, keepdims=True)
        v1 = jnp.max(p, axis=-1, keepdims=True)
        p2 = jnp.where(p >= v1, -jnp.inf, p)
        v2 = jnp.max(p2, axis=-1, keepdims=True)
        s = v1 + v2 + 1e-20
        comb_ref[...] = jnp.where(p >= v1, v1 / s,
                                  jnp.where(p >= v2, v2 / s, 0.0))
        out_ref[...] = jnp.zeros((T, H), jnp.float32)
        xb_ref[...] = x32.astype(jnp.bfloat16)

    xb = xb_ref[...]

    @pl.when(e < E)
    def _routed():
        wgb = wg_ref[0].astype(jnp.bfloat16)
        wub = wu_ref[0].astype(jnp.bfloat16)
        wdb = wd_ref[0].astype(jnp.bfloat16)
        g = jnp.dot(xb, wgb, preferred_element_type=jnp.float32)
        u = jnp.dot(xb, wub, preferred_element_type=jnp.float32)
        lane = jax.lax.broadcasted_iota(jnp.int32, (T, E), 1)
        col = jnp.sum(jnp.where(lane == e, comb_ref[...], 0.0),
                      axis=-1, keepdims=True)
        h = _silu_mul(g, u) * col
        out_ref[...] += jnp.dot(h.astype(jnp.bfloat16), wdb,
                                 preferred_element_type=jnp.float32)

    @pl.when(e == E)
    def _shared():
        sgb = swg_ref[...].astype(jnp.bfloat16)
        sub = swu_ref[...].astype(jnp.bfloat16)
        sdb = swd_ref[...].astype(jnp.bfloat16)
        g = jnp.dot(xb, sgb, preferred_element_type=jnp.float32)
        u = jnp.dot(xb, sub, preferred_element_type=jnp.float32)
        h = _silu_mul(g, u)
        out_ref[...] += jnp.dot(h.astype(jnp.bfloat16), sdb,
                                 preferred_element_type=jnp.float32)


def kernel(hidden_states, gate_w, w_gate, w_up, w_down, sw_gate, sw_up, sw_down):
    b, s, h = hidden_states.shape
    x = hidden_states.reshape(s, h)

    out = pl.pallas_call(
        _moe_body,
        grid=(E + 1,),
        in_specs=[
            pl.BlockSpec((T, H), lambda e: (0, 0)),
            pl.BlockSpec((E, H), lambda e: (0, 0)),
            pl.BlockSpec((1, H, I_DIM), lambda e: (jnp.minimum(e, E - 1), 0, 0)),
            pl.BlockSpec((1, H, I_DIM), lambda e: (jnp.minimum(e, E - 1), 0, 0)),
            pl.BlockSpec((1, I_DIM, H), lambda e: (jnp.minimum(e, E - 1), 0, 0)),
            pl.BlockSpec((H, IS_DIM), lambda e: (0, 0)),
            pl.BlockSpec((H, IS_DIM), lambda e: (0, 0)),
            pl.BlockSpec((IS_DIM, H), lambda e: (0, 0)),
        ],
        out_specs=pl.BlockSpec((T, H), lambda e: (0, 0)),
        out_shape=jax.ShapeDtypeStruct((s, h), jnp.float32),
        scratch_shapes=[
            pltpu.VMEM((T, E), jnp.float32),
            pltpu.VMEM((T, H), jnp.bfloat16),
        ],
    )(x, gate_w, w_gate, w_up, w_down, sw_gate, sw_up, sw_down)
    return out.reshape(b, s, h)
